# Initial kernel scaffold; baseline (speedup 1.0000x reference)
import functools

import jax
import jax.numpy as jnp
from jax import lax
from jax.experimental import pallas as pl
from jax.experimental.pallas import tpu as pltpu
from jax.experimental.pallas import tpu_sc as plsc

N = 10000
D = 128
E = 320000
NUM_LAYERS = 3

NC = 2
NS = 16
LANES = 16

H = D // NC
NPAD = 10240
R = NPAD // NS
CH = 128
ET = E // NS
K = 80
B = ET // K


def _zero_buf(ref, nrows):
    zvec = jnp.zeros((LANES,), jnp.float32)

    def body(r, carry):
        for j in range(H // LANES):
            ref[r, pl.ds(j * LANES, LANES)] = zvec
        return carry

    lax.fori_loop(0, nrows, body, 0)


def _gcn_body(x_hbm, col_hbm, row_hbm, val_hbm, out_hbm, work_hbm,
              col2d, row2d, val2d, gbuf, bufA, runsum, acc, sem0, sem1):
    c = lax.axis_index("c")
    s = lax.axis_index("s")

    pltpu.sync_copy(col_hbm.at[s], col2d)
    pltpu.sync_copy(row_hbm.at[s], row2d)
    pltpu.sync_copy(val_hbm.at[s], val2d)

    base_vec = jnp.full((LANES,), c * NPAD, dtype=jnp.int32)

    def add_base(b, carry):
        for j in range(K // LANES):
            sl = pl.ds(j * LANES, LANES)
            col2d[b, sl] = col2d[b, sl] + base_vec
        return carry

    lax.fori_loop(0, B, add_base, 0)

    _zero_buf(runsum, R)
    _zero_buf(bufA, CH)
    row0 = s * R
    for ch in range(R // CH):
        pltpu.sync_copy(bufA, acc.at[pl.ds(row0 + ch * CH, CH)])
    plsc.subcore_barrier()

    for layer in range(NUM_LAYERS):
        src = x_hbm if layer == 0 else work_hbm

        def pair(i, carry):
            b0 = 2 * i
            b1 = 2 * i + 1
            cp0 = pltpu.async_copy(src.at[col2d.at[b0]], gbuf.at[0], sem0)
            cp1 = pltpu.async_copy(src.at[col2d.at[b1]], gbuf.at[1], sem1)
            for slot, b, cp in ((0, b0, cp0), (1, b1, cp1)):
                cp.wait()

                def scale(e, carry2):
                    v = val2d[b, e]
                    for j in range(H // LANES):
                        sl = pl.ds(j * LANES, LANES)
                        gbuf[slot, e, sl] = gbuf[slot, e, sl] * v
                    return carry2

                lax.fori_loop(0, K, scale, 0)
                pltpu.sync_copy(gbuf.at[slot], acc.at[row2d.at[b]], add=True)
            return carry

        lax.fori_loop(0, B // 2, pair, 0)
        plsc.subcore_barrier()

        for ch in range(R // CH):
            r0 = row0 + ch * CH
            pltpu.sync_copy(acc.at[pl.ds(r0, CH)], bufA)

            def accum(r, carry):
                for j in range(H // LANES):
                    sl = pl.ds(j * LANES, LANES)
                    runsum[ch * CH + r, sl] = runsum[ch * CH + r, sl] + bufA[r, sl]
                return carry

            lax.fori_loop(0, CH, accum, 0)
            if layer < NUM_LAYERS - 1:
                pltpu.sync_copy(bufA, work_hbm.at[pl.ds(c * NPAD + r0, CH)])
                _zero_buf(bufA, CH)
                pltpu.sync_copy(bufA, acc.at[pl.ds(r0, CH)])
        plsc.subcore_barrier()

    inv = 1.0 / (NUM_LAYERS + 1)
    for ch in range(R // CH):
        r0 = row0 + ch * CH
        pltpu.sync_copy(x_hbm.at[pl.ds(c * NPAD + r0, CH)], bufA)

        def fin(r, carry):
            for j in range(H // LANES):
                sl = pl.ds(j * LANES, LANES)
                bufA[r, sl] = (bufA[r, sl] + runsum[ch * CH + r, sl]) * inv
            return carry

        lax.fori_loop(0, CH, fin, 0)
        pltpu.sync_copy(bufA, out_hbm.at[pl.ds(c * NPAD + r0, CH)])


_mesh = plsc.VectorSubcoreMesh(
    core_axis_name="c", subcore_axis_name="s", num_cores=NC, num_subcores=NS)

_gcn = functools.partial(
    pl.kernel,
    out_type=[
        jax.ShapeDtypeStruct((NC * NPAD, H), jnp.float32),
        jax.ShapeDtypeStruct((NC * NPAD, H), jnp.float32),
    ],
    mesh=_mesh,
    scratch_types=[
        pltpu.VMEM((B, K), jnp.int32),
        pltpu.VMEM((B, K), jnp.int32),
        pltpu.VMEM((B, K), jnp.float32),
        pltpu.VMEM((2, K, H), jnp.float32),
        pltpu.VMEM((CH, H), jnp.float32),
        pltpu.VMEM((R, H), jnp.float32),
        pltpu.VMEM_SHARED((NPAD, H), jnp.float32),
        pltpu.SemaphoreType.DMA,
        pltpu.SemaphoreType.DMA,
    ],
)(_gcn_body)


@jax.jit
def kernel(x, adj_index, adj_values):
    x = x.astype(jnp.float32)
    row = adj_index[0].astype(jnp.int32)
    col = adj_index[1].astype(jnp.int32)
    vals = adj_values.astype(jnp.float32)

    xp = jnp.pad(x, ((0, NPAD - N), (0, 0)))
    x_flat = jnp.concatenate([xp[:, :H], xp[:, H:]], axis=0)
    col3 = col.reshape(NS, B, K)
    row3 = row.reshape(NS, B, K)
    val3 = vals.reshape(NS, B, K)

    out_flat, _ = _gcn(x_flat, col3, row3, val3)
    out2 = out_flat.reshape(NC, NPAD, H)
    return jnp.concatenate([out2[0, :N], out2[1, :N]], axis=1)


# speedup vs baseline: 5.2341x; 5.2341x over previous
import functools

import jax
import jax.numpy as jnp
from jax import lax
from jax.experimental import pallas as pl
from jax.experimental.pallas import tpu as pltpu
from jax.experimental.pallas import tpu_sc as plsc

N = 10000
D = 128
E = 320000
NUM_LAYERS = 3

NC = 2
NS = 16
LANES = 16

H = D // NC
NPAD = 10240
R = NPAD // NS
CH = 128
ET = E // NS
K = 80
B = ET // K


def _zero_buf(ref, nrows):
    zvec = jnp.zeros((LANES,), jnp.float32)

    def body(r, carry):
        for j in range(H // LANES):
            ref[r, pl.ds(j * LANES, LANES)] = zvec
        return carry

    lax.fori_loop(0, nrows, body, 0)


def _gcn_body(x_hbm, col_hbm, row_hbm, val_hbm, out_hbm, work_hbm,
              col2d, row2d, val2d, gbuf, bufA, bufB, acc, sem0, sem1):
    c = lax.axis_index("c")
    s = lax.axis_index("s")

    pltpu.sync_copy(col_hbm.at[s], col2d)
    pltpu.sync_copy(row_hbm.at[s], row2d)
    pltpu.sync_copy(val_hbm.at[s], val2d)

    base_vec = jnp.full((LANES,), c * NPAD, dtype=jnp.int32)

    def add_base(b, carry):
        for j in range(K // LANES):
            sl = pl.ds(j * LANES, LANES)
            col2d[b, sl] = col2d[b, sl] + base_vec
        return carry

    lax.fori_loop(0, B, add_base, 0)

    _zero_buf(bufA, CH)
    row0 = s * R
    for ch in range(R // CH):
        r0 = row0 + ch * CH
        pltpu.sync_copy(bufA, acc.at[pl.ds(r0, CH)])
        pltpu.sync_copy(x_hbm.at[pl.ds(c * NPAD + r0, CH)], bufB)
        pltpu.sync_copy(bufB, out_hbm.at[pl.ds(c * NPAD + r0, CH)])
    plsc.subcore_barrier()

    for layer in range(NUM_LAYERS):
        src = x_hbm if layer == 0 else work_hbm

        def pair(i, carry):
            b0 = 2 * i
            b1 = 2 * i + 1
            cp0 = pltpu.async_copy(src.at[col2d.at[b0]], gbuf.at[0], sem0)
            cp1 = pltpu.async_copy(src.at[col2d.at[b1]], gbuf.at[1], sem1)
            for slot, b, cp in ((0, b0, cp0), (1, b1, cp1)):
                cp.wait()

                def scale(g, carry2):
                    vvec = val2d[b, pl.ds(g * LANES, LANES)]
                    for e16 in range(LANES):
                        v = vvec[e16]
                        for j in range(H // LANES):
                            sl = pl.ds(j * LANES, LANES)
                            gbuf[slot, g * LANES + e16, sl] = (
                                gbuf[slot, g * LANES + e16, sl] * v)
                    return carry2

                lax.fori_loop(0, K // LANES, scale, 0)
                pltpu.sync_copy(gbuf.at[slot], acc.at[row2d.at[b]], add=True)
            return carry

        lax.fori_loop(0, B // 2, pair, 0)
        plsc.subcore_barrier()

        for ch in range(R // CH):
            r0 = row0 + ch * CH
            pltpu.sync_copy(acc.at[pl.ds(r0, CH)], bufA)
            pltpu.sync_copy(out_hbm.at[pl.ds(c * NPAD + r0, CH)], bufB)

            def accum(r, carry):
                for j in range(H // LANES):
                    sl = pl.ds(j * LANES, LANES)
                    bufB[r, sl] = bufB[r, sl] + bufA[r, sl]
                return carry

            lax.fori_loop(0, CH, accum, 0)
            pltpu.sync_copy(bufB, out_hbm.at[pl.ds(c * NPAD + r0, CH)])
            if layer < NUM_LAYERS - 1:
                pltpu.sync_copy(bufA, work_hbm.at[pl.ds(c * NPAD + r0, CH)])
                _zero_buf(bufA, CH)
                pltpu.sync_copy(bufA, acc.at[pl.ds(r0, CH)])
        plsc.subcore_barrier()

    inv = 1.0 / (NUM_LAYERS + 1)
    for ch in range(R // CH):
        r0 = row0 + ch * CH
        pltpu.sync_copy(out_hbm.at[pl.ds(c * NPAD + r0, CH)], bufB)

        def fin(r, carry):
            for j in range(H // LANES):
                sl = pl.ds(j * LANES, LANES)
                bufB[r, sl] = bufB[r, sl] * inv
            return carry

        lax.fori_loop(0, CH, fin, 0)
        pltpu.sync_copy(bufB, out_hbm.at[pl.ds(c * NPAD + r0, CH)])


_mesh = plsc.VectorSubcoreMesh(
    core_axis_name="c", subcore_axis_name="s", num_cores=NC, num_subcores=NS)

_gcn = functools.partial(
    pl.kernel,
    out_type=[
        jax.ShapeDtypeStruct((NC * NPAD, H), jnp.float32),
        jax.ShapeDtypeStruct((NC * NPAD, H), jnp.float32),
    ],
    mesh=_mesh,
    compiler_params=pltpu.CompilerParams(use_tc_tiling_on_sc=False),
    scratch_types=[
        pltpu.VMEM((B, K), jnp.int32),
        pltpu.VMEM((B, K), jnp.int32),
        pltpu.VMEM((B, K), jnp.float32),
        pltpu.VMEM((2, K, H), jnp.float32),
        pltpu.VMEM((CH, H), jnp.float32),
        pltpu.VMEM((CH, H), jnp.float32),
        pltpu.VMEM_SHARED((NPAD, H), jnp.float32),
        pltpu.SemaphoreType.DMA,
        pltpu.SemaphoreType.DMA,
    ],
)(_gcn_body)


@jax.jit
def kernel(x, adj_index, adj_values):
    x = x.astype(jnp.float32)
    row = adj_index[0].astype(jnp.int32)
    col = adj_index[1].astype(jnp.int32)
    vals = adj_values.astype(jnp.float32)

    xp = jnp.pad(x, ((0, NPAD - N), (0, 0)))
    x_flat = jnp.concatenate([xp[:, :H], xp[:, H:]], axis=0)
    col3 = col.reshape(NS, B, K)
    row3 = row.reshape(NS, B, K)
    val3 = vals.reshape(NS, B, K)

    out_flat, _ = _gcn(x_flat, col3, row3, val3)
    out2 = out_flat.reshape(NC, NPAD, H)
    return jnp.concatenate([out2[0, :N], out2[1, :N]], axis=1)
